# Initial kernel scaffold; baseline (speedup 1.0000x reference)
import jax
import jax.numpy as jnp
from jax.experimental import pallas as pl


def kernel(id1, W):
    raise NotImplementedError("write your pallas kernel here")


# speedup vs baseline: 5.0676x; 5.0676x over previous
import functools

import jax
import jax.numpy as jnp
from jax import lax
from jax.experimental import pallas as pl
from jax.experimental.pallas import tpu as pltpu
from jax.experimental.pallas import tpu_sc as plsc

_ROWS, _COLS, _EDIM = 16384, 200, 5
_N_IN = _ROWS * _COLS
_N_OUT = _N_IN * _EDIM
_L = 16
_NC, _NS = 2, 16
_NW = _NC * _NS
_IN_PER_W = _N_IN // _NW
_CHUNK = 2048
_NCHUNK = _IN_PER_W // _CHUNK
_GROUPS = _CHUNK // _L


@functools.partial(
    pl.kernel,
    out_type=jax.ShapeDtypeStruct((_N_OUT,), jnp.float32),
    mesh=plsc.VectorSubcoreMesh(core_axis_name="c", subcore_axis_name="s"),
    compiler_params=pltpu.CompilerParams(needs_layout_passes=False),
    scratch_types=[
        pltpu.VMEM((_CHUNK,), jnp.int32),
        pltpu.VMEM((_CHUNK * _EDIM,), jnp.float32),
        pltpu.VMEM((2 * _EDIM * _L,), jnp.float32),
    ],
)
def _sc_lookup(x_hbm, pat_hbm, out_hbm, x_v, o_v, pat_v):
    wid = lax.axis_index("s") * _NC + lax.axis_index("c")
    pltpu.sync_copy(pat_hbm, pat_v)
    w0 = [pat_v[pl.ds(_L * k, _L)] for k in range(_EDIM)]
    w1 = [pat_v[pl.ds(_L * (_EDIM + k), _L)] for k in range(_EDIM)]
    iota5 = lax.iota(jnp.int32, _L) * _EDIM
    in_base = wid * _IN_PER_W

    def chunk_body(ci, carry):
        off = in_base + ci * _CHUNK
        pltpu.sync_copy(x_hbm.at[pl.ds(off, _CHUNK)], x_v)

        def grp(t, c):
            xv = x_v[pl.ds(t * _L, _L)]
            m = xv == 0
            base = t * (_EDIM * _L)
            for k in range(_EDIM):
                vals = jnp.where(m, w0[k], w1[k])
                plsc.store_scatter(o_v, [iota5 + (base + k)], vals)
            return c

        lax.fori_loop(0, _GROUPS, grp, 0)
        pltpu.sync_copy(o_v, out_hbm.at[pl.ds(off * _EDIM, _CHUNK * _EDIM)])
        return carry

    lax.fori_loop(0, _NCHUNK, chunk_body, 0)


def kernel(id1, W):
    x = id1.reshape(_N_IN)
    pat = jnp.concatenate([jnp.repeat(W[0], _L), jnp.repeat(W[1], _L)])
    out = _sc_lookup(x, pat)
    return out.reshape(_ROWS, _COLS, _EDIM)
